# baseline (speedup 1.0000x reference)
import functools

import jax
import jax.numpy as jnp
from jax import lax
from jax.experimental import pallas as pl
from jax.experimental.pallas import tpu as pltpu
from jax.experimental.pallas import tpu_sc as plsc

NC = 2
NS = 16
NW = NC * NS
L = 16
C = 32
NBUF = 3


@functools.lru_cache(maxsize=None)
def _build(B, V, F, D):
    BPW = B // NW
    n_groups = BPW // C

    mesh = plsc.VectorSubcoreMesh(core_axis_name="c", subcore_axis_name="s")

    @functools.partial(
        pl.kernel,
        mesh=mesh,
        out_type=jax.ShapeDtypeStruct((B, D), jnp.float32),
        compiler_params=pltpu.CompilerParams(needs_layout_passes=False),
        scratch_types=[
            pltpu.VMEM((BPW,), jnp.int32),
            pltpu.VMEM((BPW,), jnp.int32),
            pltpu.VMEM((BPW,), jnp.int32),
            pltpu.VMEM_SHARED((F, D), jnp.float32),
            pltpu.VMEM((NBUF, C, D), jnp.float32),
            pltpu.SemaphoreType.DMA((NBUF,)),
            pltpu.SemaphoreType.DMA,
            pltpu.SemaphoreType.DMA,
        ],
    )
    def sc_kernel(fpi_hbm, fbi_hbm, val_hbm, fpm_hbm, fbt_hbm, out_hbm,
                  fpi_v, fbi_v, val_v, fb_sh, rows,
                  sem_g, sem_w, sem_m):
        wid = lax.axis_index("s") * NC + lax.axis_index("c")
        base = wid * BPW
        pltpu.sync_copy(fpi_hbm.at[pl.ds(base, BPW)], fpi_v)
        pltpu.sync_copy(fbi_hbm.at[pl.ds(base, BPW)], fbi_v)
        pltpu.sync_copy(val_hbm.at[pl.ds(base, BPW)], val_v)
        cp_stage = pltpu.async_copy(fbt_hbm, fb_sh, sem_m)

        def gather(g, b):
            off = g * C
            pltpu.async_copy(fpm_hbm.at[fpi_v.at[pl.ds(off, C)]],
                             rows.at[b], sem_g.at[b])

        gather(0, 0)
        cp_stage.wait()
        gather(1, 1)

        def step(g, carry):
            off = g * C
            b = lax.rem(g, NBUF)

            @pl.when(g > 0)
            def _():
                pltpu.make_async_copy(rows.at[b],
                                      out_hbm.at[pl.ds(base, C)],
                                      sem_w).wait()

            @pl.when(g + 2 < n_groups)
            def _():
                gather(jnp.minimum(g + 2, n_groups - 1),
                       lax.rem(g + 2, NBUF))

            pltpu.make_async_copy(fpm_hbm.at[fpi_v.at[pl.ds(off, C)]],
                                  rows.at[b], sem_g.at[b]).wait()

            buf = rows.at[b]
            for h in range(C // L):
                val16 = val_v[pl.ds(off + h * L, L)]
                fbi16 = fbi_v[pl.ds(off + h * L, L)]
                for e in range(L):
                    @pl.when(val16[e] == 0)
                    def _(e=e, h=h, val16=val16, fbi16=fbi16):
                        pltpu.async_copy(fb_sh.at[fbi16[e]],
                                         buf.at[h * L + e], sem_m)
            for h in range(C // L):
                val16 = val_v[pl.ds(off + h * L, L)]
                for e in range(L):
                    @pl.when(val16[e] == 0)
                    def _(e=e, h=h, val16=val16):
                        pltpu.make_async_copy(fb_sh.at[0],
                                              buf.at[h * L + e],
                                              sem_m).wait()

            pltpu.async_copy(buf, out_hbm.at[pl.ds(base + off, C)], sem_w)
            return carry

        lax.fori_loop(0, n_groups, step, 0)

        pltpu.make_async_copy(rows.at[lax.rem(jnp.int32(n_groups - 1), NBUF)],
                              out_hbm.at[pl.ds(base, C)], sem_w).wait()

    return sc_kernel


def kernel(fp_idx, fb_idx, is_valid, fingerprint_matrix, fallback_table):
    B = fp_idx.shape[0]
    D = fingerprint_matrix.shape[1]
    sc = _build(B, fingerprint_matrix.shape[0], fallback_table.shape[0], D)
    return sc(fp_idx.astype(jnp.int32),
              fb_idx.astype(jnp.int32),
              is_valid.astype(jnp.int32),
              fingerprint_matrix,
              fallback_table)


# speedup vs baseline: 1.0388x; 1.0388x over previous
import functools

import jax
import jax.numpy as jnp
from jax import lax
from jax.experimental import pallas as pl
from jax.experimental.pallas import tpu as pltpu
from jax.experimental.pallas import tpu_sc as plsc

NC = 2
NS = 16
NW = NC * NS
L = 16
NBUF = 4


@functools.lru_cache(maxsize=None)
def _build(B, V, F, D):
    BPW = B // NW
    n_groups = BPW // L

    mesh = plsc.VectorSubcoreMesh(core_axis_name="c", subcore_axis_name="s")

    @functools.partial(
        pl.kernel,
        mesh=mesh,
        out_type=jax.ShapeDtypeStruct((B, D), jnp.float32),
        compiler_params=pltpu.CompilerParams(needs_layout_passes=False),
        scratch_types=[
            pltpu.VMEM((BPW,), jnp.int32),
            pltpu.VMEM((BPW,), jnp.int32),
            pltpu.VMEM((BPW,), jnp.int32),
            pltpu.VMEM_SHARED((F, D), jnp.float32),
            pltpu.VMEM((NBUF, L, D), jnp.float32),
            pltpu.SemaphoreType.DMA((NBUF,)),
            pltpu.SemaphoreType.DMA,
            pltpu.SemaphoreType.DMA,
        ],
    )
    def sc_kernel(fpi_hbm, fbi_hbm, val_hbm, fpm_hbm, fbt_hbm, out_hbm,
                  fpi_v, fbi_v, val_v, fb_sh, rows,
                  sem_g, sem_w, sem_m):
        wid = lax.axis_index("s") * NC + lax.axis_index("c")
        base = wid * BPW
        pltpu.sync_copy(fpi_hbm.at[pl.ds(base, BPW)], fpi_v)
        pltpu.sync_copy(fbi_hbm.at[pl.ds(base, BPW)], fbi_v)
        pltpu.sync_copy(val_hbm.at[pl.ds(base, BPW)], val_v)
        cp_stage = pltpu.async_copy(fbt_hbm, fb_sh, sem_m)

        def gather(g, b):
            off = g * L
            pltpu.async_copy(fpm_hbm.at[fpi_v.at[pl.ds(off, L)]],
                             rows.at[b], sem_g.at[b])

        gather(0, 0)
        cp_stage.wait()
        gather(1, 1)

        def step(g, carry):
            off = g * L
            b = lax.rem(g, NBUF)

            @pl.when(g > 1)
            def _():
                pltpu.make_async_copy(rows.at[b],
                                      out_hbm.at[pl.ds(base, L)],
                                      sem_w).wait()

            @pl.when(g + 2 < n_groups)
            def _():
                gather(jnp.minimum(g + 2, n_groups - 1),
                       lax.rem(g + 2, NBUF))

            pltpu.make_async_copy(fpm_hbm.at[fpi_v.at[pl.ds(off, L)]],
                                  rows.at[b], sem_g.at[b]).wait()

            val16 = val_v[pl.ds(off, L)]
            fbi16 = fbi_v[pl.ds(off, L)]
            buf = rows.at[b]

            for e in range(L):
                @pl.when(val16[e] == 0)
                def _(e=e):
                    pltpu.async_copy(fb_sh.at[fbi16[e]], buf.at[e], sem_m)

            for e in range(L):
                @pl.when(val16[e] == 0)
                def _(e=e):
                    pltpu.make_async_copy(fb_sh.at[0], buf.at[e],
                                          sem_m).wait()

            pltpu.async_copy(buf, out_hbm.at[pl.ds(base + off, L)], sem_w)
            return carry

        lax.fori_loop(0, n_groups, step, 0)

        pltpu.make_async_copy(rows.at[0], out_hbm.at[pl.ds(base, L)],
                              sem_w).wait()
        pltpu.make_async_copy(rows.at[1], out_hbm.at[pl.ds(base, L)],
                              sem_w).wait()

    return sc_kernel


def kernel(fp_idx, fb_idx, is_valid, fingerprint_matrix, fallback_table):
    B = fp_idx.shape[0]
    D = fingerprint_matrix.shape[1]
    sc = _build(B, fingerprint_matrix.shape[0], fallback_table.shape[0], D)
    return sc(fp_idx.astype(jnp.int32),
              fb_idx.astype(jnp.int32),
              is_valid.astype(jnp.int32),
              fingerprint_matrix,
              fallback_table)
